# baseline (speedup 1.0000x reference)
import functools

import jax
import jax.numpy as jnp
from jax import lax
from jax.experimental import pallas as pl
from jax.experimental.pallas import tpu as pltpu
from jax.experimental.pallas import tpu_sc as plsc

B, R, C = 16, 2048, 512
S = 1024
RS = R - S
NW = 32
ROWS_W = RS // 2
CHUNK_R = 32
NCH = ROWS_W // CHUNK_R
BR = 256

_mesh = plsc.VectorSubcoreMesh(core_axis_name="c", subcore_axis_name="s")


@functools.partial(
    pl.kernel,
    mesh=_mesh,
    out_type=jax.ShapeDtypeStruct((NW * 32,), jnp.float32),
    scratch_types=[
        pltpu.VMEM((2, CHUNK_R, C), jnp.float32),
        pltpu.VMEM((2, CHUNK_R // 4, C), jnp.int32),
        pltpu.VMEM((32,), jnp.float32),
        pltpu.SemaphoreType.DMA,
        pltpu.SemaphoreType.DMA,
    ],
    compiler_params=pltpu.CompilerParams(use_tc_tiling_on_sc=True),
)
def _masked_sum_sc(mi_hbm, inp_hbm, out_hbm, ibuf, mbuf, obuf, sem0, sem1):
    core = lax.axis_index("c")
    sub = lax.axis_index("s")
    wid = sub * 2 + core
    b = sub
    r_base = S + core * ROWS_W
    m_base = core * (ROWS_W // 4)
    sems = (sem0, sem1)

    def copies(g, slot):
        r0 = r_base + g * CHUNK_R
        m0 = m_base + g * (CHUNK_R // 4)
        return (
            pltpu.make_async_copy(inp_hbm.at[b, pl.ds(r0, CHUNK_R), :], ibuf.at[slot], sems[slot]),
            pltpu.make_async_copy(mi_hbm.at[b, pl.ds(m0, CHUNK_R // 4), :], mbuf.at[slot], sems[slot]),
        )

    def start(g, slot):
        for cp in copies(g, slot):
            cp.start()

    def wait(g, slot):
        for cp in copies(g, slot):
            cp.wait()

    def compute(slot, acc):
        def rowgrp_body(r4, carry):
            s, cnt = carry
            rb = pl.multiple_of(r4 * 4, 4)
            for cb in range(C // 16):
                mw = mbuf[slot, r4, pl.ds(cb * 16, 16)]
                for j in range(4):
                    mf = ((mw >> (8 * j)) & 1).astype(jnp.float32)
                    v = ibuf[slot, rb + j, pl.ds(cb * 16, 16)]
                    s = s + v * mf
                    cnt = cnt + mf
            return (s, cnt)

        return lax.fori_loop(0, CHUNK_R // 4, rowgrp_body, acc)

    start(0, 0)
    start(1, 1)
    acc = (jnp.zeros((16,), jnp.float32), jnp.zeros((16,), jnp.float32))

    def main_body(G, carry):
        for slot in range(2):
            g = 2 * G + slot
            wait(g, slot)
            carry = compute(slot, carry)
            start(g + 2, slot)
        return carry

    acc = lax.fori_loop(0, NCH // 2 - 1, main_body, acc)
    for slot in range(2):
        g = NCH - 2 + slot
        wait(g, slot)
        acc = compute(slot, acc)

    obuf[pl.ds(0, 16)] = acc[0]
    obuf[pl.ds(16, 16)] = acc[1]
    pltpu.sync_copy(obuf, out_hbm.at[pl.ds(wid * 32, 32)])


def _tc_body(m_ref, x_ref, o_ref, acc_ref):
    bi = pl.program_id(0)
    ri = pl.program_id(1)

    @pl.when((bi == 0) & (ri == 0))
    def _():
        acc_ref[...] = jnp.zeros_like(acc_ref)

    m = m_ref[0]
    x = x_ref[0]
    sel = jnp.where(m, x, 0.0).reshape(BR // 8, 8, 4, 128)
    cntf = m.astype(jnp.float32).reshape(BR // 8, 8, 4, 128)
    acc_ref[0] += sel.sum(axis=(0, 2))
    acc_ref[1] += cntf.sum(axis=(0, 2))

    @pl.when((bi == pl.num_programs(0) - 1) & (ri == pl.num_programs(1) - 1))
    def _():
        o_ref[...] = acc_ref[...]


_masked_sum_tc = pl.pallas_call(
    _tc_body,
    grid=(B, S // BR),
    in_specs=[
        pl.BlockSpec((1, BR, C), lambda b, i: (b, i, 0)),
        pl.BlockSpec((1, BR, C), lambda b, i: (b, i, 0)),
    ],
    out_specs=pl.BlockSpec((2, 8, 128), lambda b, i: (0, 0, 0)),
    out_shape=jax.ShapeDtypeStruct((2, 8, 128), jnp.float32),
    scratch_shapes=[pltpu.VMEM((2, 8, 128), jnp.float32)],
)


def kernel(mask, input):
    mi = (
        mask[:, S + 0 :: 4, :].astype(jnp.int32)
        | (mask[:, S + 1 :: 4, :].astype(jnp.int32) << 8)
        | (mask[:, S + 2 :: 4, :].astype(jnp.int32) << 16)
        | (mask[:, S + 3 :: 4, :].astype(jnp.int32) << 24)
    )

    parts = _masked_sum_sc(mi, input).reshape(NW, 2, 16)
    tc = _masked_sum_tc(mask, input)
    total = parts[:, 0, :].sum() + tc[0].sum()
    count = parts[:, 1, :].sum() + tc[1].sum()
    return total / count


# speedup vs baseline: 1.7686x; 1.7686x over previous
import functools

import jax
import jax.numpy as jnp
from jax import lax
from jax.experimental import pallas as pl
from jax.experimental.pallas import tpu as pltpu
from jax.experimental.pallas import tpu_sc as plsc

B, R, C = 16, 2048, 512
S = 1024
RS = R - S
NW = 32
ROWS_W = RS // 2
CHUNK_R = 32
NCH = ROWS_W // CHUNK_R
BR = 256

_mesh = plsc.VectorSubcoreMesh(core_axis_name="c", subcore_axis_name="s")


@functools.partial(
    pl.kernel,
    mesh=_mesh,
    out_type=jax.ShapeDtypeStruct((NW * 32,), jnp.float32),
    scratch_types=[
        pltpu.VMEM((2, CHUNK_R, C), jnp.float32),
        pltpu.VMEM((2, CHUNK_R // 4, C), jnp.int32),
        pltpu.VMEM((32,), jnp.float32),
        pltpu.SemaphoreType.DMA,
        pltpu.SemaphoreType.DMA,
    ],
    compiler_params=pltpu.CompilerParams(use_tc_tiling_on_sc=True),
)
def _masked_sum_sc(mi_hbm, inp_hbm, out_hbm, ibuf, mbuf, obuf, sem0, sem1):
    core = lax.axis_index("c")
    sub = lax.axis_index("s")
    wid = sub * 2 + core
    b = sub
    r_base = S + core * ROWS_W
    m_base = core * (ROWS_W // 4)
    sems = (sem0, sem1)

    def copies(g, slot):
        r0 = r_base + g * CHUNK_R
        m0 = m_base + g * (CHUNK_R // 4)
        return (
            pltpu.make_async_copy(inp_hbm.at[b, pl.ds(r0, CHUNK_R), :], ibuf.at[slot], sems[slot]),
            pltpu.make_async_copy(mi_hbm.at[b, pl.ds(m0, CHUNK_R // 4), :], mbuf.at[slot], sems[slot]),
        )

    def start(g, slot):
        for cp in copies(g, slot):
            cp.start()

    def wait(g, slot):
        for cp in copies(g, slot):
            cp.wait()

    def compute(slot, acc):
        def rowgrp_body(r4, carry):
            s, cnt = carry
            rb = pl.multiple_of(r4 * 4, 4)
            for cb in range(C // 16):
                mw = mbuf[slot, r4, pl.ds(cb * 16, 16)]
                for j in range(4):
                    mf = ((mw >> (8 * j)) & 1).astype(jnp.float32)
                    v = ibuf[slot, rb + j, pl.ds(cb * 16, 16)]
                    s = s + v * mf
                    cnt = cnt + mf
            return (s, cnt)

        return lax.fori_loop(0, CHUNK_R // 4, rowgrp_body, acc)

    start(0, 0)
    start(1, 1)
    acc = (jnp.zeros((16,), jnp.float32), jnp.zeros((16,), jnp.float32))

    def main_body(G, carry):
        for slot in range(2):
            g = 2 * G + slot
            wait(g, slot)
            carry = compute(slot, carry)
            start(g + 2, slot)
        return carry

    acc = lax.fori_loop(0, NCH // 2 - 1, main_body, acc)
    for slot in range(2):
        g = NCH - 2 + slot
        wait(g, slot)
        acc = compute(slot, acc)

    obuf[pl.ds(0, 16)] = acc[0]
    obuf[pl.ds(16, 16)] = acc[1]
    pltpu.sync_copy(obuf, out_hbm.at[pl.ds(wid * 32, 32)])


def _tc_body(m_ref, x_ref, o_ref, acc_ref):
    bi = pl.program_id(0)
    ri = pl.program_id(1)

    @pl.when((bi == 0) & (ri == 0))
    def _():
        acc_ref[...] = jnp.zeros_like(acc_ref)

    a0 = jnp.zeros((8, C), jnp.float32)
    a1 = jnp.zeros((8, C), jnp.float32)
    for r in range(0, BR, 8):
        m = m_ref[0, r : r + 8, :]
        x = x_ref[0, r : r + 8, :]
        a0 = a0 + jnp.where(m, x, 0.0)
        a1 = a1 + m.astype(jnp.float32)
    acc_ref[0] += a0
    acc_ref[1] += a1

    @pl.when((bi == pl.num_programs(0) - 1) & (ri == pl.num_programs(1) - 1))
    def _():
        o_ref[...] = acc_ref[...]


_masked_sum_tc = pl.pallas_call(
    _tc_body,
    grid=(B, S // BR),
    in_specs=[
        pl.BlockSpec((1, BR, C), lambda b, i: (b, i, 0)),
        pl.BlockSpec((1, BR, C), lambda b, i: (b, i, 0)),
    ],
    out_specs=pl.BlockSpec((2, 8, C), lambda b, i: (0, 0, 0)),
    out_shape=jax.ShapeDtypeStruct((2, 8, C), jnp.float32),
    scratch_shapes=[pltpu.VMEM((2, 8, C), jnp.float32)],
)

def _repack_body(m_ref, o_ref):
    rw = lax.broadcasted_iota(jnp.int32, (32, 128), 0)
    rr = lax.broadcasted_iota(jnp.int32, (32, 128), 1)
    w_lo = jnp.where(rr == 4 * rw, 1.0, 0.0) + jnp.where(rr == 4 * rw + 1, 256.0, 0.0)
    w_hi = jnp.where(rr == 4 * rw + 2, 1.0, 0.0) + jnp.where(rr == 4 * rw + 3, 256.0, 0.0)
    mf = m_ref[0].astype(jnp.float32)
    lo = jnp.dot(w_lo, mf, preferred_element_type=jnp.float32)
    hi = jnp.dot(w_hi, mf, preferred_element_type=jnp.float32)
    o_ref[0] = lo.astype(jnp.int32) | (hi.astype(jnp.int32) << 16)


_repack_tc = pl.pallas_call(
    _repack_body,
    grid=(B, RS // 128),
    in_specs=[pl.BlockSpec((1, 128, C), lambda b, i: (b, (S // 128) + i, 0))],
    out_specs=pl.BlockSpec((1, 32, C), lambda b, i: (b, i, 0)),
    out_shape=jax.ShapeDtypeStruct((B, RS // 4, C), jnp.int32),
)


def kernel(mask, input):
    mi = _repack_tc(mask)
    parts = _masked_sum_sc(mi, input).reshape(NW, 2, 16)
    tc = _masked_sum_tc(mask, input)
    total = parts[:, 0, :].sum() + tc[0].sum()
    count = parts[:, 1, :].sum() + tc[1].sum()
    return total / count


# speedup vs baseline: 3.2787x; 1.8538x over previous
import functools

import jax
import jax.numpy as jnp
from jax import lax
from jax.experimental import pallas as pl
from jax.experimental.pallas import tpu as pltpu
from jax.experimental.pallas import tpu_sc as plsc

B, R, C = 16, 2048, 512
S = 1024
RS = R - S
NW = 32
ROWS_W = RS // 2
CHUNK_R = 32
NCH = ROWS_W // CHUNK_R
BR = 512

_mesh = plsc.VectorSubcoreMesh(core_axis_name="c", subcore_axis_name="s")


@functools.partial(
    pl.kernel,
    mesh=_mesh,
    out_type=jax.ShapeDtypeStruct((NW * 32,), jnp.float32),
    scratch_types=[
        pltpu.VMEM((2, CHUNK_R, C), jnp.float32),
        pltpu.VMEM((2, CHUNK_R // 4, C), jnp.int32),
        pltpu.VMEM((32,), jnp.float32),
        pltpu.SemaphoreType.DMA,
        pltpu.SemaphoreType.DMA,
    ],
    compiler_params=pltpu.CompilerParams(use_tc_tiling_on_sc=True),
)
def _masked_sum_sc(mi_hbm, inp_hbm, out_hbm, ibuf, mbuf, obuf, sem0, sem1):
    core = lax.axis_index("c")
    sub = lax.axis_index("s")
    wid = sub * 2 + core
    b = sub
    r_base = S + core * ROWS_W
    m_base = core * (ROWS_W // 4)
    sems = (sem0, sem1)

    def copies(g, slot):
        r0 = r_base + g * CHUNK_R
        m0 = m_base + g * (CHUNK_R // 4)
        return (
            pltpu.make_async_copy(inp_hbm.at[b, pl.ds(r0, CHUNK_R), :], ibuf.at[slot], sems[slot]),
            pltpu.make_async_copy(mi_hbm.at[b, pl.ds(m0, CHUNK_R // 4), :], mbuf.at[slot], sems[slot]),
        )

    def start(g, slot):
        for cp in copies(g, slot):
            cp.start()

    def wait(g, slot):
        for cp in copies(g, slot):
            cp.wait()

    def compute(slot, acc):
        def rowgrp_body(r4, carry):
            s, cnt = carry
            rb = pl.multiple_of(r4 * 4, 4)
            for cb in range(C // 16):
                mw = mbuf[slot, r4, pl.ds(cb * 16, 16)]
                for j in range(4):
                    mf = ((mw >> (8 * j)) & 1).astype(jnp.float32)
                    v = ibuf[slot, rb + j, pl.ds(cb * 16, 16)]
                    s = s + v * mf
                    cnt = cnt + mf
            return (s, cnt)

        return lax.fori_loop(0, CHUNK_R // 4, rowgrp_body, acc)

    start(0, 0)
    start(1, 1)
    acc = (jnp.zeros((16,), jnp.float32), jnp.zeros((16,), jnp.float32))

    def main_body(G, carry):
        for slot in range(2):
            g = 2 * G + slot
            wait(g, slot)
            carry = compute(slot, carry)
            start(g + 2, slot)
        return carry

    acc = lax.fori_loop(0, NCH // 2 - 1, main_body, acc)
    for slot in range(2):
        g = NCH - 2 + slot
        wait(g, slot)
        acc = compute(slot, acc)

    obuf[pl.ds(0, 16)] = acc[0]
    obuf[pl.ds(16, 16)] = acc[1]
    pltpu.sync_copy(obuf, out_hbm.at[pl.ds(wid * 32, 32)])


def _tc_body(m_ref, x_ref, o_ref, acc_ref):
    bi = pl.program_id(0)
    ri = pl.program_id(1)

    @pl.when((bi == 0) & (ri == 0))
    def _():
        acc_ref[...] = jnp.zeros_like(acc_ref)

    a0 = jnp.zeros((8, C), jnp.float32)
    a1 = jnp.zeros((8, C), jnp.float32)
    for r in range(0, BR, 8):
        m = m_ref[0, r : r + 8, :] != 0
        x = x_ref[0, r : r + 8, :]
        a0 = a0 + jnp.where(m, x, 0.0)
        a1 = a1 + m.astype(jnp.float32)
    acc_ref[0] += a0
    acc_ref[1] += a1

    @pl.when((bi == pl.num_programs(0) - 1) & (ri == pl.num_programs(1) - 1))
    def _():
        o_ref[...] = acc_ref[...]


_masked_sum_tc = pl.pallas_call(
    _tc_body,
    grid=(B, S // BR),
    in_specs=[
        pl.BlockSpec((1, BR, C), lambda b, i: (b, i, 0)),
        pl.BlockSpec((1, BR, C), lambda b, i: (b, i, 0)),
    ],
    out_specs=pl.BlockSpec((2, 8, C), lambda b, i: (0, 0, 0)),
    out_shape=jax.ShapeDtypeStruct((2, 8, C), jnp.float32),
    scratch_shapes=[pltpu.VMEM((2, 8, C), jnp.float32)],
)

def _repack_body(m_ref, o_ref):
    rw = lax.broadcasted_iota(jnp.int32, (128, 512), 0)
    rr = lax.broadcasted_iota(jnp.int32, (128, 512), 1)
    w_lo = jnp.where(rr == 4 * rw, 1.0, 0.0) + jnp.where(rr == 4 * rw + 1, 256.0, 0.0)
    w_hi = jnp.where(rr == 4 * rw + 2, 1.0, 0.0) + jnp.where(rr == 4 * rw + 3, 256.0, 0.0)
    mf = (m_ref[0] != 0).astype(jnp.float32)
    lo = jnp.dot(w_lo, mf, preferred_element_type=jnp.float32)
    hi = jnp.dot(w_hi, mf, preferred_element_type=jnp.float32)
    o_ref[0] = lo.astype(jnp.int32) | (hi.astype(jnp.int32) << 16)


_repack_tc = pl.pallas_call(
    _repack_body,
    grid=(B, RS // 512),
    in_specs=[pl.BlockSpec((1, 512, C), lambda b, i: (b, (S // 512) + i, 0))],
    out_specs=pl.BlockSpec((1, 128, C), lambda b, i: (b, i, 0)),
    out_shape=jax.ShapeDtypeStruct((B, RS // 4, C), jnp.int32),
)


def kernel(mask, input):
    mask_u8 = mask.view(jnp.uint8)
    mi = _repack_tc(mask_u8)
    parts = _masked_sum_sc(mi, input).reshape(NW, 2, 16)
    tc = _masked_sum_tc(mask_u8, input)
    total = parts[:, 0, :].sum() + tc[0].sum()
    count = parts[:, 1, :].sum() + tc[1].sum()
    return total / count


# speedup vs baseline: 3.3372x; 1.0178x over previous
import functools

import jax
import jax.numpy as jnp
from jax import lax
from jax.experimental import pallas as pl
from jax.experimental.pallas import tpu as pltpu
from jax.experimental.pallas import tpu_sc as plsc

B, R, C = 16, 2048, 512
S = 1024
RS = R - S
NW = 32
ROWS_W = RS // 2
CHUNK_R = 32
NCH = ROWS_W // CHUNK_R
BR = 512

_mesh = plsc.VectorSubcoreMesh(core_axis_name="c", subcore_axis_name="s")


@functools.partial(
    pl.kernel,
    mesh=_mesh,
    out_type=jax.ShapeDtypeStruct((NW * 32,), jnp.float32),
    scratch_types=[
        pltpu.VMEM((2, CHUNK_R, C), jnp.float32),
        pltpu.VMEM((2, CHUNK_R // 4, C), jnp.int32),
        pltpu.VMEM((32,), jnp.float32),
        pltpu.SemaphoreType.DMA,
        pltpu.SemaphoreType.DMA,
    ],
    compiler_params=pltpu.CompilerParams(use_tc_tiling_on_sc=True),
)
def _masked_sum_sc(mi_hbm, inp_hbm, out_hbm, ibuf, mbuf, obuf, sem0, sem1):
    core = lax.axis_index("c")
    sub = lax.axis_index("s")
    wid = sub * 2 + core
    b = sub
    r_base = S + core * ROWS_W
    m_base = core * (ROWS_W // 4)
    sems = (sem0, sem1)

    def copies(g, slot):
        r0 = r_base + g * CHUNK_R
        m0 = m_base + g * (CHUNK_R // 4)
        return (
            pltpu.make_async_copy(inp_hbm.at[b, pl.ds(r0, CHUNK_R), :], ibuf.at[slot], sems[slot]),
            pltpu.make_async_copy(mi_hbm.at[b, pl.ds(m0, CHUNK_R // 4), :], mbuf.at[slot], sems[slot]),
        )

    def start(g, slot):
        for cp in copies(g, slot):
            cp.start()

    def wait(g, slot):
        for cp in copies(g, slot):
            cp.wait()

    def compute(slot, acc):
        def rowgrp_body(r4, carry):
            s, cnt = carry
            rb = pl.multiple_of(r4 * 4, 4)
            for cb in range(C // 16):
                mw = mbuf[slot, r4, pl.ds(cb * 16, 16)]
                for j in range(4):
                    mf = ((mw >> (8 * j)) & 1).astype(jnp.float32)
                    v = ibuf[slot, rb + j, pl.ds(cb * 16, 16)]
                    s = s + v * mf
                    cnt = cnt + mf
            return (s, cnt)

        return lax.fori_loop(0, CHUNK_R // 4, rowgrp_body, acc)

    start(0, 0)
    start(1, 1)
    acc = (jnp.zeros((16,), jnp.float32), jnp.zeros((16,), jnp.float32))

    def main_body(G, carry):
        for slot in range(2):
            g = 2 * G + slot
            wait(g, slot)
            carry = compute(slot, carry)
            start(g + 2, slot)
        return carry

    acc = lax.fori_loop(0, NCH // 2 - 1, main_body, acc)
    for slot in range(2):
        g = NCH - 2 + slot
        wait(g, slot)
        acc = compute(slot, acc)

    obuf[pl.ds(0, 16)] = acc[0]
    obuf[pl.ds(16, 16)] = acc[1]
    pltpu.sync_copy(obuf, out_hbm.at[pl.ds(wid * 32, 32)])


def _tc_body(m_ref, x_ref, o_ref, acc_ref):
    bi = pl.program_id(0)
    ri = pl.program_id(1)

    @pl.when((bi == 0) & (ri == 0))
    def _():
        acc_ref[...] = jnp.zeros_like(acc_ref)

    a0 = jnp.zeros((8, C), jnp.float32)
    a1 = jnp.zeros((8, C), jnp.float32)
    for r in range(0, BR, 8):
        mf = m_ref[0, r : r + 8, :].astype(jnp.float32)
        x = x_ref[0, r : r + 8, :]
        a0 = a0 + x * mf
        a1 = a1 + mf
    acc_ref[0] += a0
    acc_ref[1] += a1

    @pl.when((bi == pl.num_programs(0) - 1) & (ri == pl.num_programs(1) - 1))
    def _():
        o_ref[...] = acc_ref[...]


_masked_sum_tc = pl.pallas_call(
    _tc_body,
    grid=(B, S // BR),
    in_specs=[
        pl.BlockSpec((1, BR, C), lambda b, i: (b, i, 0)),
        pl.BlockSpec((1, BR, C), lambda b, i: (b, i, 0)),
    ],
    out_specs=pl.BlockSpec((2, 8, C), lambda b, i: (0, 0, 0)),
    out_shape=jax.ShapeDtypeStruct((2, 8, C), jnp.float32),
    scratch_shapes=[pltpu.VMEM((2, 8, C), jnp.float32)],
)

def _repack_body(m_ref, o_ref, w_ref):
    @pl.when((pl.program_id(0) == 0) & (pl.program_id(1) == 0))
    def _():
        rw = lax.broadcasted_iota(jnp.int32, (128, 512), 0)
        rr = lax.broadcasted_iota(jnp.int32, (128, 512), 1)
        w_ref[0] = jnp.where(rr == 4 * rw, 1.0, 0.0) + jnp.where(rr == 4 * rw + 1, 256.0, 0.0)
        w_ref[1] = jnp.where(rr == 4 * rw + 2, 1.0, 0.0) + jnp.where(rr == 4 * rw + 3, 256.0, 0.0)

    mf = m_ref[0].astype(jnp.float32)
    lo = jnp.dot(w_ref[0], mf, preferred_element_type=jnp.float32)
    hi = jnp.dot(w_ref[1], mf, preferred_element_type=jnp.float32)
    o_ref[0] = lo.astype(jnp.int32) | (hi.astype(jnp.int32) << 16)


_repack_tc = pl.pallas_call(
    _repack_body,
    grid=(B, RS // 512),
    in_specs=[pl.BlockSpec((1, 512, C), lambda b, i: (b, (S // 512) + i, 0))],
    out_specs=pl.BlockSpec((1, 128, C), lambda b, i: (b, i, 0)),
    out_shape=jax.ShapeDtypeStruct((B, RS // 4, C), jnp.int32),
    scratch_shapes=[pltpu.VMEM((2, 128, 512), jnp.float32)],
)


def kernel(mask, input):
    mask_u8 = mask.view(jnp.uint8)
    mi = _repack_tc(mask_u8)
    parts = _masked_sum_sc(mi, input).reshape(NW, 2, 16)
    tc = _masked_sum_tc(mask_u8, input)
    total = parts[:, 0, :].sum() + tc[0].sum()
    count = parts[:, 1, :].sum() + tc[1].sum()
    return total / count


# speedup vs baseline: 3.4428x; 1.0317x over previous
import functools

import jax
import jax.numpy as jnp
from jax import lax
from jax.experimental import pallas as pl
from jax.experimental.pallas import tpu as pltpu
from jax.experimental.pallas import tpu_sc as plsc

B, R, C = 16, 2048, 512
S = 1024
RS = R - S
NW = 32
ROWS_W = RS // 2
CHUNK_R = 64
NCH = ROWS_W // CHUNK_R
BR = 512

_mesh = plsc.VectorSubcoreMesh(core_axis_name="c", subcore_axis_name="s")


@functools.partial(
    pl.kernel,
    mesh=_mesh,
    out_type=jax.ShapeDtypeStruct((NW * 32,), jnp.float32),
    scratch_types=[
        pltpu.VMEM((2, CHUNK_R, C), jnp.float32),
        pltpu.VMEM((2, CHUNK_R // 4, C), jnp.int32),
        pltpu.VMEM((32,), jnp.float32),
        pltpu.SemaphoreType.DMA,
        pltpu.SemaphoreType.DMA,
    ],
    compiler_params=pltpu.CompilerParams(use_tc_tiling_on_sc=True),
)
def _masked_sum_sc(mi_hbm, inp_hbm, out_hbm, ibuf, mbuf, obuf, sem0, sem1):
    core = lax.axis_index("c")
    sub = lax.axis_index("s")
    wid = sub * 2 + core
    b = sub
    r_base = S + core * ROWS_W
    m_base = core * (ROWS_W // 4)
    sems = (sem0, sem1)

    def copies(g, slot):
        r0 = r_base + g * CHUNK_R
        m0 = m_base + g * (CHUNK_R // 4)
        return (
            pltpu.make_async_copy(inp_hbm.at[b, pl.ds(r0, CHUNK_R), :], ibuf.at[slot], sems[slot]),
            pltpu.make_async_copy(mi_hbm.at[b, pl.ds(m0, CHUNK_R // 4), :], mbuf.at[slot], sems[slot]),
        )

    def start(g, slot):
        for cp in copies(g, slot):
            cp.start()

    def wait(g, slot):
        for cp in copies(g, slot):
            cp.wait()

    def compute(slot, acc):
        def rowgrp_body(r4, carry):
            s, cnt = carry
            rb = pl.multiple_of(r4 * 4, 4)
            for cb in range(C // 16):
                mw = mbuf[slot, r4, pl.ds(cb * 16, 16)]
                for j in range(4):
                    mf = ((mw >> (8 * j)) & 1).astype(jnp.float32)
                    v = ibuf[slot, rb + j, pl.ds(cb * 16, 16)]
                    s = s + v * mf
                    cnt = cnt + mf
            return (s, cnt)

        return lax.fori_loop(0, CHUNK_R // 4, rowgrp_body, acc)

    start(0, 0)
    start(1, 1)
    acc = (jnp.zeros((16,), jnp.float32), jnp.zeros((16,), jnp.float32))

    def main_body(G, carry):
        for slot in range(2):
            g = 2 * G + slot
            wait(g, slot)
            carry = compute(slot, carry)
            start(g + 2, slot)
        return carry

    acc = lax.fori_loop(0, NCH // 2 - 1, main_body, acc)
    for slot in range(2):
        g = NCH - 2 + slot
        wait(g, slot)
        acc = compute(slot, acc)

    obuf[pl.ds(0, 16)] = acc[0]
    obuf[pl.ds(16, 16)] = acc[1]
    pltpu.sync_copy(obuf, out_hbm.at[pl.ds(wid * 32, 32)])


def _tc_body(m_ref, x_ref, o_ref, acc_ref):
    bi = pl.program_id(0)
    ri = pl.program_id(1)

    @pl.when((bi == 0) & (ri == 0))
    def _():
        acc_ref[...] = jnp.zeros_like(acc_ref)

    a0 = jnp.zeros((8, C), jnp.float32)
    a1 = jnp.zeros((8, C), jnp.float32)
    for r in range(0, BR, 8):
        mf = m_ref[0, r : r + 8, :].astype(jnp.float32)
        x = x_ref[0, r : r + 8, :]
        a0 = a0 + x * mf
        a1 = a1 + mf
    acc_ref[0] += a0
    acc_ref[1] += a1

    @pl.when((bi == pl.num_programs(0) - 1) & (ri == pl.num_programs(1) - 1))
    def _():
        o_ref[...] = acc_ref[...]


_masked_sum_tc = pl.pallas_call(
    _tc_body,
    grid=(B, S // BR),
    in_specs=[
        pl.BlockSpec((1, BR, C), lambda b, i: (b, i, 0)),
        pl.BlockSpec((1, BR, C), lambda b, i: (b, i, 0)),
    ],
    out_specs=pl.BlockSpec((2, 8, C), lambda b, i: (0, 0, 0)),
    out_shape=jax.ShapeDtypeStruct((2, 8, C), jnp.float32),
    scratch_shapes=[pltpu.VMEM((2, 8, C), jnp.float32)],
)

def _repack_body(m_ref, o_ref, w_ref):
    @pl.when((pl.program_id(0) == 0) & (pl.program_id(1) == 0))
    def _():
        rw = lax.broadcasted_iota(jnp.int32, (128, 512), 0)
        rr = lax.broadcasted_iota(jnp.int32, (128, 512), 1)
        w_ref[0] = jnp.where(rr == 4 * rw, 1.0, 0.0) + jnp.where(rr == 4 * rw + 1, 256.0, 0.0)
        w_ref[1] = jnp.where(rr == 4 * rw + 2, 1.0, 0.0) + jnp.where(rr == 4 * rw + 3, 256.0, 0.0)

    mf = m_ref[0].astype(jnp.float32)
    lo = jnp.dot(w_ref[0], mf, preferred_element_type=jnp.float32)
    hi = jnp.dot(w_ref[1], mf, preferred_element_type=jnp.float32)
    o_ref[0] = lo.astype(jnp.int32) | (hi.astype(jnp.int32) << 16)


_repack_tc = pl.pallas_call(
    _repack_body,
    grid=(B, RS // 512),
    in_specs=[pl.BlockSpec((1, 512, C), lambda b, i: (b, (S // 512) + i, 0))],
    out_specs=pl.BlockSpec((1, 128, C), lambda b, i: (b, i, 0)),
    out_shape=jax.ShapeDtypeStruct((B, RS // 4, C), jnp.int32),
    scratch_shapes=[pltpu.VMEM((2, 128, 512), jnp.float32)],
)


def kernel(mask, input):
    mask_u8 = mask.view(jnp.uint8)
    mi = _repack_tc(mask_u8)
    parts = _masked_sum_sc(mi, input).reshape(NW, 2, 16)
    tc = _masked_sum_tc(mask_u8, input)
    total = parts[:, 0, :].sum() + tc[0].sum()
    count = parts[:, 1, :].sum() + tc[1].sum()
    return total / count
